# Initial kernel scaffold; baseline (speedup 1.0000x reference)
import jax
import jax.numpy as jnp
from jax.experimental import pallas as pl


def kernel(x, edge_index, W_self1, W_neigh1, b1, W_self2, W_neigh2, b2):
    raise NotImplementedError("write your pallas kernel here")


# speedup vs baseline: 5.1809x; 5.1809x over previous
import functools

import jax
import jax.numpy as jnp
from jax import lax
from jax.experimental import pallas as pl
from jax.experimental.pallas import tpu as pltpu
from jax.experimental.pallas import tpu_sc as plsc

N = 10000
D = 128
E = 320000

NC = 2
NS = 16
NW = NC * NS
EDGES_PER_TILE = E // NW
CHUNK = 80
N_ITERS = EDGES_PER_TILE // CHUNK
ROWS_PER_TILE = 624
TAIL_ROWS = N - ROWS_PER_TILE * NS


def _sc_agg_body(x_hbm, src_hbm, dst_hbm, z2_hbm,
                 s_out, cnt_out,
                 src_idx, dst_idx, rows, ones_v, cbuf, sem, s_acc, cnt_acc):
    c = lax.axis_index("c")
    s = lax.axis_index("s")
    wid = c * NS + s

    for i in range(CHUNK // 16):
        ones_v[pl.ds(i * 16, 16)] = jnp.full((16,), 1.0, jnp.float32)
    for i in range(ROWS_PER_TILE // 16):
        cbuf[pl.ds(i * 16, 16)] = jnp.zeros((16,), jnp.float32)

    r0 = pl.multiple_of(s * ROWS_PER_TILE, 8)
    pltpu.sync_copy(z2_hbm.at[pl.ds(r0, ROWS_PER_TILE)],
                    s_acc.at[pl.ds(r0, ROWS_PER_TILE)])
    pltpu.sync_copy(cbuf, cnt_acc.at[pl.ds(r0, ROWS_PER_TILE)])

    @pl.when(s == NS - 1)
    def _():
        t0 = ROWS_PER_TILE * NS
        pltpu.sync_copy(z2_hbm.at[pl.ds(t0, TAIL_ROWS)],
                        s_acc.at[pl.ds(t0, TAIL_ROWS)])
        pltpu.sync_copy(cbuf.at[pl.ds(0, TAIL_ROWS)],
                        cnt_acc.at[pl.ds(t0, TAIL_ROWS)])

    plsc.subcore_barrier()

    e0 = wid * EDGES_PER_TILE

    def body(i, carry):
        base = pl.multiple_of(e0 + i * CHUNK, 8)
        pltpu.sync_copy(src_hbm.at[pl.ds(base, CHUNK)], src_idx)
        pltpu.sync_copy(dst_hbm.at[pl.ds(base, CHUNK)], dst_idx)
        pltpu.async_copy(x_hbm.at[src_idx], rows, sem).wait()
        pltpu.sync_copy(rows, s_acc.at[dst_idx], add=True)
        pltpu.sync_copy(ones_v, cnt_acc.at[dst_idx], add=True)
        return carry

    lax.fori_loop(0, N_ITERS, body, 0)

    plsc.subcore_barrier()

    o0 = pl.multiple_of(c * N + r0, 8)
    pltpu.sync_copy(s_acc.at[pl.ds(r0, ROWS_PER_TILE)],
                    s_out.at[pl.ds(o0, ROWS_PER_TILE)])
    pltpu.sync_copy(cnt_acc.at[pl.ds(r0, ROWS_PER_TILE)], cbuf)
    pltpu.sync_copy(cbuf, cnt_out.at[pl.ds(o0, ROWS_PER_TILE)])

    @pl.when(s == NS - 1)
    def _():
        t0 = ROWS_PER_TILE * NS
        ot = pl.multiple_of(c * N + t0, 8)
        pltpu.sync_copy(s_acc.at[pl.ds(t0, TAIL_ROWS)],
                        s_out.at[pl.ds(ot, TAIL_ROWS)])
        pltpu.sync_copy(cnt_acc.at[pl.ds(t0, TAIL_ROWS)],
                        cbuf.at[pl.ds(0, TAIL_ROWS)])
        pltpu.sync_copy(cbuf.at[pl.ds(0, TAIL_ROWS)],
                        cnt_out.at[pl.ds(ot, TAIL_ROWS)])


@functools.cache
def _sc_agg_kernel():
    return functools.partial(
        pl.kernel,
        mesh=plsc.VectorSubcoreMesh(core_axis_name="c", subcore_axis_name="s"),
        out_type=[
            jax.ShapeDtypeStruct((NC * N, D), jnp.float32),
            jax.ShapeDtypeStruct((NC * N,), jnp.float32),
        ],
        scratch_types=[
            pltpu.VMEM((CHUNK,), jnp.int32),
            pltpu.VMEM((CHUNK,), jnp.int32),
            pltpu.VMEM((CHUNK, D), jnp.float32),
            pltpu.VMEM((CHUNK,), jnp.float32),
            pltpu.VMEM((ROWS_PER_TILE,), jnp.float32),
            pltpu.SemaphoreType.DMA,
            pltpu.VMEM_SHARED((N, D), jnp.float32),
            pltpu.VMEM_SHARED((N,), jnp.float32),
        ],
    )(_sc_agg_body)


BLK = 1000


def _combine_body(h_ref, s0_ref, s1_ref, c0_ref, c1_ref, ws_ref, wn_ref,
                  b_ref, o_ref, *, relu):
    cnt = c0_ref[...] + c1_ref[...]
    inv = 1.0 / jnp.maximum(cnt, 1.0)
    agg = (s0_ref[...] + s1_ref[...]) * inv
    acc = jnp.dot(h_ref[...], ws_ref[...], preferred_element_type=jnp.float32)
    acc = acc + jnp.dot(agg, wn_ref[...], preferred_element_type=jnp.float32)
    acc = acc + b_ref[...]
    if relu:
        acc = jnp.maximum(acc, 0.0)
    o_ref[...] = acc


def _combine(h, s2, c2, W_self, W_neigh, b, relu):
    s0 = s2[:N]
    s1 = s2[N:]
    c0 = c2[:N].reshape(N, 1)
    c1 = c2[N:].reshape(N, 1)
    return pl.pallas_call(
        functools.partial(_combine_body, relu=relu),
        grid=(N // BLK,),
        in_specs=[
            pl.BlockSpec((BLK, D), lambda i: (i, 0)),
            pl.BlockSpec((BLK, D), lambda i: (i, 0)),
            pl.BlockSpec((BLK, D), lambda i: (i, 0)),
            pl.BlockSpec((BLK, 1), lambda i: (i, 0)),
            pl.BlockSpec((BLK, 1), lambda i: (i, 0)),
            pl.BlockSpec((D, D), lambda i: (0, 0)),
            pl.BlockSpec((D, D), lambda i: (0, 0)),
            pl.BlockSpec((1, D), lambda i: (0, 0)),
        ],
        out_specs=pl.BlockSpec((BLK, D), lambda i: (i, 0)),
        out_shape=jax.ShapeDtypeStruct((N, D), jnp.float32),
    )(h, s0, s1, c0, c1, W_self, W_neigh, b.reshape(1, D))


def kernel(x, edge_index, W_self1, W_neigh1, b1, W_self2, W_neigh2, b2):
    src = edge_index[0]
    dst = edge_index[1]
    z2 = jnp.zeros((N, D), jnp.float32)

    agg_fn = _sc_agg_kernel()
    s2, c2 = agg_fn(x, src, dst, z2)
    h1 = _combine(x, s2, c2, W_self1, W_neigh1, b1, relu=True)

    s2b, c2b = agg_fn(h1, src, dst, z2)
    out = _combine(h1, s2b, c2b, W_self2, W_neigh2, b2, relu=False)
    return out


# speedup vs baseline: 9.0452x; 1.7459x over previous
import functools

import jax
import jax.numpy as jnp
from jax import lax
from jax.experimental import pallas as pl
from jax.experimental.pallas import tpu as pltpu
from jax.experimental.pallas import tpu_sc as plsc

N = 10000
D = 128
E = 320000

NC = 2
NS = 16
NW = NC * NS
EDGES_PER_TILE = E // NW
CHUNK = 40
N_ITERS = EDGES_PER_TILE // CHUNK
ROWS_PER_TILE = 624
TAIL_ROWS = N - ROWS_PER_TILE * NS


NBUF = 5
NGROUPS = N_ITERS // NBUF


def _make_sc_agg_body(with_cnt):
    def body(x_hbm, src_hbm, dst_hbm, z2_hbm, *rest):
        if with_cnt:
            (s_out, cnt_out, src_idx, dst_idx, rows, ones_v, cbuf,
             *sems, s_acc, cnt_acc) = rest
        else:
            (s_out, src_idx, dst_idx, rows, *sems, s_acc) = rest
        si = sems[:NBUF]
        sg = sems[NBUF:2 * NBUF]
        ss = sems[2 * NBUF:3 * NBUF]

        c = lax.axis_index("c")
        s = lax.axis_index("s")
        wid = c * NS + s

        if with_cnt:
            for i in range(CHUNK // 16):
                ones_v[pl.ds(i * 16, 16)] = jnp.full((16,), 1.0, jnp.float32)
            for i in range(ROWS_PER_TILE // 16):
                cbuf[pl.ds(i * 16, 16)] = jnp.zeros((16,), jnp.float32)

        r0 = pl.multiple_of(s * ROWS_PER_TILE, 8)
        pltpu.sync_copy(z2_hbm.at[pl.ds(r0, ROWS_PER_TILE)],
                        s_acc.at[pl.ds(r0, ROWS_PER_TILE)])
        if with_cnt:
            pltpu.sync_copy(cbuf, cnt_acc.at[pl.ds(r0, ROWS_PER_TILE)])

        @pl.when(s == NS - 1)
        def _():
            t0 = ROWS_PER_TILE * NS
            pltpu.sync_copy(z2_hbm.at[pl.ds(t0, TAIL_ROWS)],
                            s_acc.at[pl.ds(t0, TAIL_ROWS)])
            if with_cnt:
                pltpu.sync_copy(cbuf.at[pl.ds(0, TAIL_ROWS)],
                                cnt_acc.at[pl.ds(t0, TAIL_ROWS)])

        plsc.subcore_barrier()

        e0 = wid * EDGES_PER_TILE

        def start_idx(b, chunk):
            base = pl.multiple_of(e0 + chunk * CHUNK, 8)
            pltpu.async_copy(src_hbm.at[pl.ds(base, CHUNK)], src_idx.at[b],
                             si[b])
            pltpu.async_copy(dst_hbm.at[pl.ds(base, CHUNK)], dst_idx.at[b],
                             si[b])

        def wait_idx(b):
            pltpu.make_async_copy(src_hbm.at[pl.ds(0, CHUNK)], src_idx.at[b],
                                  si[b]).wait()
            pltpu.make_async_copy(dst_hbm.at[pl.ds(0, CHUNK)], dst_idx.at[b],
                                  si[b]).wait()

        def start_gather(b):
            pltpu.async_copy(x_hbm.at[src_idx.at[b]], rows.at[b], sg[b])

        def wait_gather(b):
            pltpu.make_async_copy(x_hbm.at[src_idx.at[b]], rows.at[b],
                                  sg[b]).wait()

        def run_group(g, start_next):
            descs = []
            for b in range(NBUF):
                wait_gather(b)
                descs.append(pltpu.async_copy(
                    rows.at[b], s_acc.at[dst_idx.at[b]], ss[b], add=True))
                if with_cnt:
                    descs.append(pltpu.async_copy(
                        ones_v, cnt_acc.at[dst_idx.at[b]], ss[b], add=True))
            for d in descs:
                d.wait()
            if start_next:
                nxt = (g + 1) * NBUF
                for b in range(NBUF):
                    start_idx(b, nxt + b)
                for b in range(NBUF):
                    wait_idx(b)
                    start_gather(b)

        for b in range(NBUF):
            start_idx(b, b)
        for b in range(NBUF):
            wait_idx(b)
            start_gather(b)

        def group(g, carry):
            run_group(g, True)
            return carry

        lax.fori_loop(0, NGROUPS - 1, group, 0)
        run_group(NGROUPS - 1, False)

        plsc.subcore_barrier()

        o0 = pl.multiple_of(c * N + r0, 8)
        pltpu.sync_copy(s_acc.at[pl.ds(r0, ROWS_PER_TILE)],
                        s_out.at[pl.ds(o0, ROWS_PER_TILE)])
        if with_cnt:
            pltpu.sync_copy(cnt_acc.at[pl.ds(r0, ROWS_PER_TILE)], cbuf)
            pltpu.sync_copy(cbuf, cnt_out.at[pl.ds(o0, ROWS_PER_TILE)])

        @pl.when(s == NS - 1)
        def _():
            t0 = ROWS_PER_TILE * NS
            ot = pl.multiple_of(c * N + t0, 8)
            pltpu.sync_copy(s_acc.at[pl.ds(t0, TAIL_ROWS)],
                            s_out.at[pl.ds(ot, TAIL_ROWS)])
            if with_cnt:
                pltpu.sync_copy(cnt_acc.at[pl.ds(t0, TAIL_ROWS)],
                                cbuf.at[pl.ds(0, TAIL_ROWS)])
                pltpu.sync_copy(cbuf.at[pl.ds(0, TAIL_ROWS)],
                                cnt_out.at[pl.ds(ot, TAIL_ROWS)])

    return body


@functools.lru_cache(maxsize=None)
def _sc_agg_kernel(with_cnt):
    out_type = [jax.ShapeDtypeStruct((NC * N, D), jnp.float32)]
    scratch = [
        pltpu.VMEM((NBUF, CHUNK), jnp.int32),
        pltpu.VMEM((NBUF, CHUNK), jnp.int32),
        pltpu.VMEM((NBUF, CHUNK, D), jnp.float32),
    ]
    if with_cnt:
        out_type.append(jax.ShapeDtypeStruct((NC * N,), jnp.float32))
        scratch.append(pltpu.VMEM((CHUNK,), jnp.float32))
        scratch.append(pltpu.VMEM((ROWS_PER_TILE,), jnp.float32))
    scratch.extend([pltpu.SemaphoreType.DMA] * (3 * NBUF))
    scratch.append(pltpu.VMEM_SHARED((N, D), jnp.float32))
    if with_cnt:
        scratch.append(pltpu.VMEM_SHARED((N,), jnp.float32))
    return functools.partial(
        pl.kernel,
        mesh=plsc.VectorSubcoreMesh(core_axis_name="c", subcore_axis_name="s"),
        out_type=out_type,
        scratch_types=scratch,
    )(_make_sc_agg_body(with_cnt))


BLK = 1000


def _combine_body(h_ref, s0_ref, s1_ref, c0_ref, c1_ref, ws_ref, wn_ref,
                  b_ref, o_ref, *, relu):
    cnt = c0_ref[...] + c1_ref[...]
    inv = 1.0 / jnp.maximum(cnt, 1.0)
    agg = (s0_ref[...] + s1_ref[...]) * inv
    acc = jnp.dot(h_ref[...], ws_ref[...], preferred_element_type=jnp.float32)
    acc = acc + jnp.dot(agg, wn_ref[...], preferred_element_type=jnp.float32)
    acc = acc + b_ref[...]
    if relu:
        acc = jnp.maximum(acc, 0.0)
    o_ref[...] = acc


def _combine(h, s2, c2, W_self, W_neigh, b, relu):
    s0 = s2[:N]
    s1 = s2[N:]
    c0 = c2[:N].reshape(N, 1)
    c1 = c2[N:].reshape(N, 1)
    return pl.pallas_call(
        functools.partial(_combine_body, relu=relu),
        grid=(N // BLK,),
        in_specs=[
            pl.BlockSpec((BLK, D), lambda i: (i, 0)),
            pl.BlockSpec((BLK, D), lambda i: (i, 0)),
            pl.BlockSpec((BLK, D), lambda i: (i, 0)),
            pl.BlockSpec((BLK, 1), lambda i: (i, 0)),
            pl.BlockSpec((BLK, 1), lambda i: (i, 0)),
            pl.BlockSpec((D, D), lambda i: (0, 0)),
            pl.BlockSpec((D, D), lambda i: (0, 0)),
            pl.BlockSpec((1, D), lambda i: (0, 0)),
        ],
        out_specs=pl.BlockSpec((BLK, D), lambda i: (i, 0)),
        out_shape=jax.ShapeDtypeStruct((N, D), jnp.float32),
    )(h, s0, s1, c0, c1, W_self, W_neigh, b.reshape(1, D))


def kernel(x, edge_index, W_self1, W_neigh1, b1, W_self2, W_neigh2, b2):
    src = edge_index[0]
    dst = edge_index[1]
    z2 = jnp.zeros((N, D), jnp.float32)

    agg = _sc_agg_kernel(True)
    s2, c2 = agg(x, src, dst, z2)
    h1 = _combine(x, s2, c2, W_self1, W_neigh1, b1, relu=True)

    s2b, _ = agg(h1, src, dst, z2)
    out = _combine(h1, s2b, c2, W_self2, W_neigh2, b2, relu=False)
    return out
